# baseline (speedup 1.0000x reference)
import functools

import jax
import jax.numpy as jnp
from jax import lax
from jax.experimental import pallas as pl
from jax.experimental.pallas import tpu as pltpu
from jax.experimental.pallas import tpu_sc as plsc

_N_U = 50000
_N_I = 50000
_D = 128
_NNZ = 600000
_L = 3
_B = 4096

_NPAD = 50176
_NCHUNK = 4
_RCHUNK = 12544
_STRIPE = 784
_TRASH = 12544
_ZROWS = 12552


_EB = 768
_NNZPAD = 614400
_NBLK = 50
_SB = 80
_NSUB = 2
_SCAP = 848


def _spmm_sc(rows, cols, vals, x):
    mesh = plsc.VectorSubcoreMesh(core_axis_name="c", subcore_axis_name="s")

    @functools.partial(
        pl.kernel,
        out_type=jax.ShapeDtypeStruct((_NPAD, _D), jnp.float32),
        mesh=mesh,
        compiler_params=pltpu.CompilerParams(needs_layout_passes=False),
        scratch_types=[
            pltpu.VMEM((_EB,), jnp.int32),
            pltpu.VMEM((_EB,), jnp.int32),
            pltpu.VMEM((_EB,), jnp.float32),
            pltpu.VMEM((_EB,), jnp.int32),
            pltpu.VMEM((_EB,), jnp.int32),
            pltpu.VMEM((_EB,), jnp.float32),
            pltpu.VMEM((_SCAP,), jnp.int32),
            pltpu.VMEM((_SCAP,), jnp.int32),
            pltpu.VMEM((_SCAP,), jnp.float32),
            pltpu.VMEM((_NSUB, _SB), jnp.int32),
            pltpu.VMEM((_NSUB, _SB), jnp.int32),
            pltpu.VMEM((_NSUB, _SB), jnp.float32),
            pltpu.VMEM((_NSUB * _SB, _D), jnp.float32),
            pltpu.VMEM_SHARED((_ZROWS, _D), jnp.float32),
            pltpu.SMEM((1,), jnp.int32),
            pltpu.SMEM((1,), jnp.int32),
            pltpu.SMEM((1,), jnp.int32),
            pltpu.SemaphoreType.DMA,
            pltpu.SemaphoreType.DMA,
            pltpu.SemaphoreType.DMA,
            pltpu.SemaphoreType.DMA,
        ],
    )
    def body(rows_hbm, cols_hbm, vals_hbm, x_hbm, z_hbm,
             rows_a, cols_a, vals_a, rows_b, cols_b, vals_b,
             st_ridx, st_cols, st_vals, ridx2d, cols2d, vals2d, gat_v, zacc,
             off_s, head_s, tail_s, esem_a, esem_b, gsem, ssem):
        c = lax.axis_index("c")
        s = lax.axis_index("s")
        i16 = lax.iota(jnp.int32, 16)
        z16 = jnp.zeros((16,), jnp.float32)
        tile_e0 = s * (_NBLK * _EB)

        def load_blk(bi, rv, cv, vv, sem):
            e0 = tile_e0 + bi * _EB
            pltpu.async_copy(rows_hbm.at[pl.ds(e0, _EB)], rv, sem)
            pltpu.async_copy(cols_hbm.at[pl.ds(e0, _EB)], cv, sem)
            pltpu.async_copy(vals_hbm.at[pl.ds(e0, _EB)], vv, sem)

        def wait_blk(bi, rv, cv, vv, sem):
            e0 = tile_e0 + bi * _EB
            pltpu.make_async_copy(rows_hbm.at[pl.ds(e0, _EB)], rv, sem).wait()
            pltpu.make_async_copy(cols_hbm.at[pl.ds(e0, _EB)], cv, sem).wait()
            pltpu.make_async_copy(vals_hbm.at[pl.ds(e0, _EB)], vv, sem).wait()

        def wait_scatter(slot):
            pltpu.make_async_copy(gat_v.at[pl.ds(slot * _SB, _SB)],
                                  zacc.at[ridx2d.at[slot]], ssem).wait()

        def fin_one():
            head = head_s[0]
            slot = lax.rem(head, _NSUB)
            gb = gat_v.at[pl.ds(slot * _SB, _SB)]
            pltpu.make_async_copy(x_hbm.at[cols2d.at[slot]], gb, gsem).wait()

            @pl.loop(0, _SB // 16)
            def _(g):
                vvec = vals2d.at[slot, pl.ds(g * 16, 16)][...]
                for j2 in range(16):
                    v = vvec[j2]
                    for k in range(_D // 16):
                        sl2 = pl.ds(k * 16, 16)
                        row = gat_v.at[slot * _SB + g * 16 + j2, sl2]
                        row[...] = row[...] * v

            pltpu.async_copy(gat_v.at[pl.ds(slot * _SB, _SB)],
                             zacc.at[ridx2d.at[slot]], ssem, add=True)
            head_s[0] = head + 1

        def issue_one(qoff):
            tail = tail_s[0]

            @pl.when(tail - head_s[0] == _NSUB)
            def _():
                fin_one()

            slot = lax.rem(tail, _NSUB)

            @pl.when(tail >= _NSUB)
            def _():
                wait_scatter(slot)

            @pl.loop(0, _SB // 16)
            def _(g):
                gsl = pl.ds(g * 16, 16)
                ssl = pl.ds(qoff + g * 16, 16)
                cols2d.at[slot, gsl][...] = st_cols.at[ssl][...]
                ridx2d.at[slot, gsl][...] = st_ridx.at[ssl][...]
                vals2d.at[slot, gsl][...] = st_vals.at[ssl][...]
            pltpu.async_copy(x_hbm.at[cols2d.at[slot]],
                             gat_v.at[pl.ds(slot * _SB, _SB)], gsem)
            tail_s[0] = tail + 1

        def drain_all():
            lax.fori_loop(head_s[0], tail_s[0],
                          lambda q, _: (fin_one(), None)[1], None)
            tail = tail_s[0]
            lo = jnp.maximum(tail - _NSUB, 0)
            lax.fori_loop(lo, tail,
                          lambda k, _: (wait_scatter(lax.rem(k, _NSUB)),
                                        None)[1], None)

        def process_blk(bi, rv, cv, vv, esem, base):
            wait_blk(bi, rv, cv, vv, esem)

            def cgroup(g, o):
                sl = pl.ds(g * 16, 16)
                rb = rv.at[sl][...] - base
                inb = lax.bitcast_convert_type(rb, jnp.uint32) < _RCHUNK
                osl = pl.ds(o, 16)
                plsc.store_compressed(st_ridx.at[osl], rb, mask=inb)
                plsc.store_compressed(st_cols.at[osl], cv.at[sl][...],
                                      mask=inb)
                plsc.store_compressed(st_vals.at[osl], vv.at[sl][...],
                                      mask=inb)
                cnt = plsc.all_reduce_population_count(inb)
                return o + cnt[0]

            t = lax.fori_loop(0, _EB // 16, cgroup, off_s[0])
            nbf = lax.div(t, _SB)

            lax.fori_loop(0, nbf,
                          lambda q, _: (issue_one(q * _SB), None)[1], None)

            rem = nbf * _SB
            for g in range(_SB // 16):
                dsl = pl.ds(g * 16, 16)
                ssl = pl.ds(rem + g * 16, 16)
                st_ridx.at[dsl][...] = st_ridx.at[ssl][...]
                st_cols.at[dsl][...] = st_cols.at[ssl][...]
                st_vals.at[dsl][...] = st_vals.at[ssl][...]
            off_s[0] = t - rem

        @pl.loop(0, _NCHUNK // 2)
        def _(p):
            base = (c * (_NCHUNK // 2) + p) * _RCHUNK

            @pl.loop(0, _SB)
            def _(r):
                for k in range(_D // 16):
                    gat_v.at[r, pl.ds(k * 16, 16)][...] = z16
            for i in range(9):
                pltpu.async_copy(gat_v.at[pl.ds(0, _SB)],
                                 zacc.at[pl.ds(s * _STRIPE + i * _SB, _SB)],
                                 esem_a)
            pltpu.sync_copy(gat_v.at[pl.ds(0, 64)],
                            zacc.at[pl.ds(s * _STRIPE + 720, 64)])
            for i in range(9):
                pltpu.make_async_copy(
                    gat_v.at[pl.ds(0, _SB)],
                    zacc.at[pl.ds(s * _STRIPE + i * _SB, _SB)],
                    esem_a).wait()
            plsc.subcore_barrier()
            off_s[0] = 0
            head_s[0] = 0
            tail_s[0] = 0

            load_blk(0, rows_a, cols_a, vals_a, esem_a)

            @pl.loop(0, _NBLK // 2)
            def _(i):
                load_blk(2 * i + 1, rows_b, cols_b, vals_b, esem_b)
                process_blk(2 * i, rows_a, cols_a, vals_a, esem_a, base)
                load_blk(lax.rem(2 * i + 2, _NBLK), rows_a, cols_a, vals_a,
                         esem_a)
                process_blk(2 * i + 1, rows_b, cols_b, vals_b, esem_b, base)

            wait_blk(0, rows_a, cols_a, vals_a, esem_a)

            t = off_s[0]
            for g in range(_SB // 16):
                sl = pl.ds(g * 16, 16)
                keep = (i16 + g * 16) < t
                st_ridx.at[sl][...] = jnp.where(keep, st_ridx.at[sl][...],
                                                _TRASH)
                st_cols.at[sl][...] = jnp.where(keep, st_cols.at[sl][...], 0)
                st_vals.at[sl][...] = jnp.where(keep, st_vals.at[sl][...],
                                                0.0)
            issue_one(0)
            drain_all()

            plsc.subcore_barrier()
            pltpu.sync_copy(zacc.at[pl.ds(s * _STRIPE, _STRIPE)],
                            z_hbm.at[pl.ds(base + s * _STRIPE, _STRIPE)])
            plsc.subcore_barrier()

    return body(rows, cols, vals, x)


def _gather3_sc(e_u, e_i, uids, pos, neg):
    mesh = plsc.VectorSubcoreMesh(core_axis_name="c", subcore_axis_name="s")
    per = _B // 32

    @functools.partial(
        pl.kernel,
        out_type=[jax.ShapeDtypeStruct((_B, _D), jnp.float32)] * 3,
        mesh=mesh,
        scratch_types=[
            pltpu.VMEM((per,), jnp.int32),
            pltpu.VMEM((per, _D), jnp.float32),
        ],
    )
    def body(eu_hbm, ei_hbm, uids_hbm, pos_hbm, neg_hbm, ou, op, on,
             idx_v, buf_v):
        c = lax.axis_index("c")
        s = lax.axis_index("s")
        off = (s * 2 + c) * per
        for ih, tab, oh in ((uids_hbm, eu_hbm, ou),
                            (pos_hbm, ei_hbm, op),
                            (neg_hbm, ei_hbm, on)):
            pltpu.sync_copy(ih.at[pl.ds(off, per)], idx_v)
            pltpu.sync_copy(tab.at[idx_v], buf_v)
            pltpu.sync_copy(buf_v, oh.at[pl.ds(off, per)])

    return body(e_u, e_i, uids, pos, neg)


def _tc_update(zp, zn, e, ws, wp, wn, bs, bp, bn):
    bm = 1568
    dn = (((1,), (1,)), ((), ()))

    def body(zp_r, zn_r, e_r, ws_r, wp_r, wn_r, bs_r, bp_r, bn_r, out_r):
        a = zp_r[...]
        b = zn_r[...]
        ee = e_r[...]
        h = (a - b
             + lax.dot_general(a * ee, wp_r[...], dn,
                               preferred_element_type=jnp.float32)
             - lax.dot_general(b * ee, wn_r[...], dn,
                               preferred_element_type=jnp.float32)
             + lax.dot_general(ee, ws_r[...], dn,
                               preferred_element_type=jnp.float32)
             + (bs_r[...] + bp_r[...] - bn_r[...]))
        out_r[...] = jnp.where(h >= 0, h, 0.2 * h)

    return pl.pallas_call(
        body,
        grid=(_NPAD // bm,),
        in_specs=[pl.BlockSpec((bm, _D), lambda i: (i, 0))] * 3
        + [pl.BlockSpec((_D, _D), lambda i: (0, 0))] * 3
        + [pl.BlockSpec((1, _D), lambda i: (0, 0))] * 3,
        out_specs=pl.BlockSpec((bm, _D), lambda i: (i, 0)),
        out_shape=jax.ShapeDtypeStruct((_NPAD, _D), jnp.float32),
    )(zp, zn, e, ws, wp, wn, bs, bp, bn)


def _tc_loss(u_emb, pos_emb, neg_emb):
    def body(u_r, p_r, n_r, ls_r, lr_r, ps_r, ns_r):
        uu = u_r[...]
        pp = p_r[...]
        nn = n_r[...]
        ps = jnp.sum(uu * pp, axis=1)
        ns = jnp.sum(uu * nn, axis=1)
        ps_r[...] = ps
        ns_r[...] = ns
        d = ps - ns
        sig = 1.0 / (1.0 + jnp.exp(-d))
        lg = jnp.clip(jnp.log(sig), -2000.0, 2000.0)
        ls_r[0, 0] = -jnp.mean(lg)
        lr_r[0, 0] = (jnp.sum(uu * uu) + jnp.sum(pp * pp)
                      + jnp.sum(nn * nn))

    return pl.pallas_call(
        body,
        out_shape=[
            jax.ShapeDtypeStruct((1, 1), jnp.float32),
            jax.ShapeDtypeStruct((1, 1), jnp.float32),
            jax.ShapeDtypeStruct((_B,), jnp.float32),
            jax.ShapeDtypeStruct((_B,), jnp.float32),
        ],
        out_specs=[
            pl.BlockSpec(memory_space=pltpu.SMEM),
            pl.BlockSpec(memory_space=pltpu.SMEM),
            pl.BlockSpec(),
            pl.BlockSpec(),
        ],
    )(u_emb, pos_emb, neg_emb)


def kernel(uids, pos, neg, pos_rows, pos_cols, pos_vals,
           neg_rows, neg_cols, neg_vals, E_u_0, E_i_0,
           Wself_w, Wself_b, Wpos_w, Wpos_b, Wneg_w, Wneg_b):
    e_u = jnp.pad(E_u_0, ((0, _NPAD - _N_U), (0, 0)))
    e_i = jnp.pad(E_i_0, ((0, _NPAD - _N_I), (0, 0)))
    padn = _NNZPAD - _NNZ
    pr = jnp.pad(pos_rows, (0, padn))
    pc = jnp.pad(pos_cols, (0, padn))
    pv = jnp.pad(pos_vals, (0, padn))
    nr = jnp.pad(neg_rows, (0, padn))
    nc = jnp.pad(neg_cols, (0, padn))
    nv = jnp.pad(neg_vals, (0, padn))

    for layer in range(_L):
        ws = Wself_w[layer]
        wp = Wpos_w[layer]
        wn = Wneg_w[layer]
        bs = Wself_b[layer].reshape(1, _D)
        bp = Wpos_b[layer].reshape(1, _D)
        bn = Wneg_b[layer].reshape(1, _D)
        z_u_pos = _spmm_sc(pr, pc, pv, e_i)
        z_u_neg = _spmm_sc(nr, nc, nv, e_i)
        z_i_pos = _spmm_sc(pc, pr, pv, e_u)
        e_u_new = _tc_update(z_u_pos, z_u_neg, e_u, ws, wp, wn, bs, bp, bn)
        z_i_neg = _spmm_sc(nc, nr, nv, e_u)
        e_i = _tc_update(z_i_pos, z_i_neg, e_i, ws, wp, wn, bs, bp, bn)
        e_u = e_u_new

    u_emb, pos_emb, neg_emb = _gather3_sc(e_u, e_i, uids, pos, neg)
    ls, lr, ps, ns = _tc_loss(u_emb, pos_emb, neg_emb)
    return (ls[0, 0], lr[0, 0], ps, ns)


# speedup vs baseline: 1.1062x; 1.1062x over previous
import functools

import jax
import jax.numpy as jnp
from jax import lax
from jax.experimental import pallas as pl
from jax.experimental.pallas import tpu as pltpu
from jax.experimental.pallas import tpu_sc as plsc

_N_U = 50000
_N_I = 50000
_D = 128
_NNZ = 600000
_L = 3
_B = 4096

_NPAD = 50176
_NCHUNK = 4
_RCHUNK = 12544
_STRIPE = 784
_TRASH = 12544
_ZROWS = 12552


_EB = 384
_NNZPAD = 602112
_NBLK = 98
_SB = 64
_NSUB = 3
_SCAP = 448


def _spmm_sc(rows, cols, vals, x):
    mesh = plsc.VectorSubcoreMesh(core_axis_name="c", subcore_axis_name="s")

    @functools.partial(
        pl.kernel,
        out_type=jax.ShapeDtypeStruct((_NPAD, _D), jnp.float32),
        mesh=mesh,
        compiler_params=pltpu.CompilerParams(needs_layout_passes=False),
        scratch_types=[
            pltpu.VMEM((_EB,), jnp.int32),
            pltpu.VMEM((_EB,), jnp.int32),
            pltpu.VMEM((_EB,), jnp.float32),
            pltpu.VMEM((_EB,), jnp.int32),
            pltpu.VMEM((_EB,), jnp.int32),
            pltpu.VMEM((_EB,), jnp.float32),
            pltpu.VMEM((_SCAP,), jnp.int32),
            pltpu.VMEM((_SCAP,), jnp.int32),
            pltpu.VMEM((_SCAP,), jnp.float32),
            pltpu.VMEM((_NSUB, _SB), jnp.int32),
            pltpu.VMEM((_NSUB, _SB), jnp.int32),
            pltpu.VMEM((_NSUB, _SB), jnp.float32),
            pltpu.VMEM((_NSUB * _SB, _D), jnp.float32),
            pltpu.VMEM_SHARED((_ZROWS, _D), jnp.float32),
            pltpu.SMEM((1,), jnp.int32),
            pltpu.SMEM((1,), jnp.int32),
            pltpu.SMEM((1,), jnp.int32),
            pltpu.SemaphoreType.DMA,
            pltpu.SemaphoreType.DMA,
            pltpu.SemaphoreType.DMA,
            pltpu.SemaphoreType.DMA,
        ],
    )
    def body(rows_hbm, cols_hbm, vals_hbm, x_hbm, z_hbm,
             rows_a, cols_a, vals_a, rows_b, cols_b, vals_b,
             st_ridx, st_cols, st_vals, ridx2d, cols2d, vals2d, gat_v, zacc,
             off_s, head_s, tail_s, esem_a, esem_b, gsem, ssem):
        c = lax.axis_index("c")
        s = lax.axis_index("s")
        i16 = lax.iota(jnp.int32, 16)
        z16 = jnp.zeros((16,), jnp.float32)
        tile_e0 = s * (_NBLK * _EB)

        def load_blk(bi, rv, cv, vv, sem):
            e0 = tile_e0 + bi * _EB
            pltpu.async_copy(rows_hbm.at[pl.ds(e0, _EB)], rv, sem)
            pltpu.async_copy(cols_hbm.at[pl.ds(e0, _EB)], cv, sem)
            pltpu.async_copy(vals_hbm.at[pl.ds(e0, _EB)], vv, sem)

        def wait_blk(bi, rv, cv, vv, sem):
            e0 = tile_e0 + bi * _EB
            pltpu.make_async_copy(rows_hbm.at[pl.ds(e0, _EB)], rv, sem).wait()
            pltpu.make_async_copy(cols_hbm.at[pl.ds(e0, _EB)], cv, sem).wait()
            pltpu.make_async_copy(vals_hbm.at[pl.ds(e0, _EB)], vv, sem).wait()

        def wait_scatter(slot):
            pltpu.make_async_copy(gat_v.at[pl.ds(slot * _SB, _SB)],
                                  zacc.at[ridx2d.at[slot]], ssem).wait()

        def fin_one():
            head = head_s[0]
            slot = lax.rem(head, _NSUB)
            gb = gat_v.at[pl.ds(slot * _SB, _SB)]
            pltpu.make_async_copy(x_hbm.at[cols2d.at[slot]], gb, gsem).wait()

            @pl.loop(0, _SB // 16)
            def _(g):
                vvec = vals2d.at[slot, pl.ds(g * 16, 16)][...]
                for j2 in range(16):
                    v = vvec[j2]
                    for k in range(_D // 16):
                        sl2 = pl.ds(k * 16, 16)
                        row = gat_v.at[slot * _SB + g * 16 + j2, sl2]
                        row[...] = row[...] * v

            pltpu.async_copy(gat_v.at[pl.ds(slot * _SB, _SB)],
                             zacc.at[ridx2d.at[slot]], ssem, add=True)
            head_s[0] = head + 1

        def issue_one(qoff):
            tail = tail_s[0]

            @pl.when(tail - head_s[0] == _NSUB)
            def _():
                fin_one()

            slot = lax.rem(tail, _NSUB)

            @pl.when(tail >= _NSUB)
            def _():
                wait_scatter(slot)

            @pl.loop(0, _SB // 16)
            def _(g):
                gsl = pl.ds(g * 16, 16)
                ssl = pl.ds(qoff + g * 16, 16)
                cols2d.at[slot, gsl][...] = st_cols.at[ssl][...]
                ridx2d.at[slot, gsl][...] = st_ridx.at[ssl][...]
                vals2d.at[slot, gsl][...] = st_vals.at[ssl][...]
            pltpu.async_copy(x_hbm.at[cols2d.at[slot]],
                             gat_v.at[pl.ds(slot * _SB, _SB)], gsem)
            tail_s[0] = tail + 1

        def drain_all():
            lax.fori_loop(head_s[0], tail_s[0],
                          lambda q, _: (fin_one(), None)[1], None)
            tail = tail_s[0]
            lo = jnp.maximum(tail - _NSUB, 0)
            lax.fori_loop(lo, tail,
                          lambda k, _: (wait_scatter(lax.rem(k, _NSUB)),
                                        None)[1], None)

        def process_blk(bi, rv, cv, vv, esem, base):
            wait_blk(bi, rv, cv, vv, esem)

            def cgroup(g, o):
                sl = pl.ds(g * 16, 16)
                rb = rv.at[sl][...] - base
                inb = lax.bitcast_convert_type(rb, jnp.uint32) < _RCHUNK
                osl = pl.ds(o, 16)
                plsc.store_compressed(st_ridx.at[osl], rb, mask=inb)
                plsc.store_compressed(st_cols.at[osl], cv.at[sl][...],
                                      mask=inb)
                plsc.store_compressed(st_vals.at[osl], vv.at[sl][...],
                                      mask=inb)
                cnt = plsc.all_reduce_population_count(inb)
                return o + cnt[0]

            t = lax.fori_loop(0, _EB // 16, cgroup, off_s[0])
            nbf = lax.div(t, _SB)

            lax.fori_loop(0, nbf,
                          lambda q, _: (issue_one(q * _SB), None)[1], None)

            rem = nbf * _SB
            for g in range(_SB // 16):
                dsl = pl.ds(g * 16, 16)
                ssl = pl.ds(rem + g * 16, 16)
                st_ridx.at[dsl][...] = st_ridx.at[ssl][...]
                st_cols.at[dsl][...] = st_cols.at[ssl][...]
                st_vals.at[dsl][...] = st_vals.at[ssl][...]
            off_s[0] = t - rem

        @pl.loop(0, _NCHUNK // 2)
        def _(p):
            base = (c * (_NCHUNK // 2) + p) * _RCHUNK

            @pl.loop(0, _SB)
            def _(r):
                for k in range(_D // 16):
                    gat_v.at[r, pl.ds(k * 16, 16)][...] = z16
            for i in range(12):
                pltpu.async_copy(gat_v.at[pl.ds(0, _SB)],
                                 zacc.at[pl.ds(s * _STRIPE + i * _SB, _SB)],
                                 esem_a)
            pltpu.sync_copy(gat_v.at[pl.ds(0, 16)],
                            zacc.at[pl.ds(s * _STRIPE + 768, 16)])
            for i in range(12):
                pltpu.make_async_copy(
                    gat_v.at[pl.ds(0, _SB)],
                    zacc.at[pl.ds(s * _STRIPE + i * _SB, _SB)],
                    esem_a).wait()
            plsc.subcore_barrier()
            off_s[0] = 0
            head_s[0] = 0
            tail_s[0] = 0

            load_blk(0, rows_a, cols_a, vals_a, esem_a)

            @pl.loop(0, _NBLK // 2)
            def _(i):
                load_blk(2 * i + 1, rows_b, cols_b, vals_b, esem_b)
                process_blk(2 * i, rows_a, cols_a, vals_a, esem_a, base)
                load_blk(lax.rem(2 * i + 2, _NBLK), rows_a, cols_a, vals_a,
                         esem_a)
                process_blk(2 * i + 1, rows_b, cols_b, vals_b, esem_b, base)

            wait_blk(0, rows_a, cols_a, vals_a, esem_a)

            t = off_s[0]
            for g in range(_SB // 16):
                sl = pl.ds(g * 16, 16)
                keep = (i16 + g * 16) < t
                st_ridx.at[sl][...] = jnp.where(keep, st_ridx.at[sl][...],
                                                _TRASH)
                st_cols.at[sl][...] = jnp.where(keep, st_cols.at[sl][...], 0)
                st_vals.at[sl][...] = jnp.where(keep, st_vals.at[sl][...],
                                                0.0)
            issue_one(0)
            drain_all()

            plsc.subcore_barrier()
            pltpu.sync_copy(zacc.at[pl.ds(s * _STRIPE, _STRIPE)],
                            z_hbm.at[pl.ds(base + s * _STRIPE, _STRIPE)])
            plsc.subcore_barrier()

    return body(rows, cols, vals, x)


def _gather3_sc(e_u, e_i, uids, pos, neg):
    mesh = plsc.VectorSubcoreMesh(core_axis_name="c", subcore_axis_name="s")
    per = _B // 32

    @functools.partial(
        pl.kernel,
        out_type=[jax.ShapeDtypeStruct((_B, _D), jnp.float32)] * 3,
        mesh=mesh,
        scratch_types=[
            pltpu.VMEM((per,), jnp.int32),
            pltpu.VMEM((per, _D), jnp.float32),
        ],
    )
    def body(eu_hbm, ei_hbm, uids_hbm, pos_hbm, neg_hbm, ou, op, on,
             idx_v, buf_v):
        c = lax.axis_index("c")
        s = lax.axis_index("s")
        off = (s * 2 + c) * per
        for ih, tab, oh in ((uids_hbm, eu_hbm, ou),
                            (pos_hbm, ei_hbm, op),
                            (neg_hbm, ei_hbm, on)):
            pltpu.sync_copy(ih.at[pl.ds(off, per)], idx_v)
            pltpu.sync_copy(tab.at[idx_v], buf_v)
            pltpu.sync_copy(buf_v, oh.at[pl.ds(off, per)])

    return body(e_u, e_i, uids, pos, neg)


def _tc_update(zp, zn, e, ws, wp, wn, bs, bp, bn):
    bm = 1568
    dn = (((1,), (1,)), ((), ()))

    def body(zp_r, zn_r, e_r, ws_r, wp_r, wn_r, bs_r, bp_r, bn_r, out_r):
        a = zp_r[...]
        b = zn_r[...]
        ee = e_r[...]
        h = (a - b
             + lax.dot_general(a * ee, wp_r[...], dn,
                               preferred_element_type=jnp.float32)
             - lax.dot_general(b * ee, wn_r[...], dn,
                               preferred_element_type=jnp.float32)
             + lax.dot_general(ee, ws_r[...], dn,
                               preferred_element_type=jnp.float32)
             + (bs_r[...] + bp_r[...] - bn_r[...]))
        out_r[...] = jnp.where(h >= 0, h, 0.2 * h)

    return pl.pallas_call(
        body,
        grid=(_NPAD // bm,),
        in_specs=[pl.BlockSpec((bm, _D), lambda i: (i, 0))] * 3
        + [pl.BlockSpec((_D, _D), lambda i: (0, 0))] * 3
        + [pl.BlockSpec((1, _D), lambda i: (0, 0))] * 3,
        out_specs=pl.BlockSpec((bm, _D), lambda i: (i, 0)),
        out_shape=jax.ShapeDtypeStruct((_NPAD, _D), jnp.float32),
    )(zp, zn, e, ws, wp, wn, bs, bp, bn)


def _tc_loss(u_emb, pos_emb, neg_emb):
    def body(u_r, p_r, n_r, ls_r, lr_r, ps_r, ns_r):
        uu = u_r[...]
        pp = p_r[...]
        nn = n_r[...]
        ps = jnp.sum(uu * pp, axis=1)
        ns = jnp.sum(uu * nn, axis=1)
        ps_r[...] = ps
        ns_r[...] = ns
        d = ps - ns
        sig = 1.0 / (1.0 + jnp.exp(-d))
        lg = jnp.clip(jnp.log(sig), -2000.0, 2000.0)
        ls_r[0, 0] = -jnp.mean(lg)
        lr_r[0, 0] = (jnp.sum(uu * uu) + jnp.sum(pp * pp)
                      + jnp.sum(nn * nn))

    return pl.pallas_call(
        body,
        out_shape=[
            jax.ShapeDtypeStruct((1, 1), jnp.float32),
            jax.ShapeDtypeStruct((1, 1), jnp.float32),
            jax.ShapeDtypeStruct((_B,), jnp.float32),
            jax.ShapeDtypeStruct((_B,), jnp.float32),
        ],
        out_specs=[
            pl.BlockSpec(memory_space=pltpu.SMEM),
            pl.BlockSpec(memory_space=pltpu.SMEM),
            pl.BlockSpec(),
            pl.BlockSpec(),
        ],
    )(u_emb, pos_emb, neg_emb)


def kernel(uids, pos, neg, pos_rows, pos_cols, pos_vals,
           neg_rows, neg_cols, neg_vals, E_u_0, E_i_0,
           Wself_w, Wself_b, Wpos_w, Wpos_b, Wneg_w, Wneg_b):
    e_u = jnp.pad(E_u_0, ((0, _NPAD - _N_U), (0, 0)))
    e_i = jnp.pad(E_i_0, ((0, _NPAD - _N_I), (0, 0)))
    padn = _NNZPAD - _NNZ
    pr = jnp.pad(pos_rows, (0, padn))
    pc = jnp.pad(pos_cols, (0, padn))
    pv = jnp.pad(pos_vals, (0, padn))
    nr = jnp.pad(neg_rows, (0, padn))
    nc = jnp.pad(neg_cols, (0, padn))
    nv = jnp.pad(neg_vals, (0, padn))

    for layer in range(_L):
        ws = Wself_w[layer]
        wp = Wpos_w[layer]
        wn = Wneg_w[layer]
        bs = Wself_b[layer].reshape(1, _D)
        bp = Wpos_b[layer].reshape(1, _D)
        bn = Wneg_b[layer].reshape(1, _D)
        z_u_pos = _spmm_sc(pr, pc, pv, e_i)
        z_u_neg = _spmm_sc(nr, nc, nv, e_i)
        z_i_pos = _spmm_sc(pc, pr, pv, e_u)
        e_u_new = _tc_update(z_u_pos, z_u_neg, e_u, ws, wp, wn, bs, bp, bn)
        z_i_neg = _spmm_sc(nc, nr, nv, e_u)
        e_i = _tc_update(z_i_pos, z_i_neg, e_i, ws, wp, wn, bs, bp, bn)
        e_u = e_u_new

    u_emb, pos_emb, neg_emb = _gather3_sc(e_u, e_i, uids, pos, neg)
    ls, lr, ps, ns = _tc_loss(u_emb, pos_emb, neg_emb)
    return (ls[0, 0], lr[0, 0], ps, ns)


# speedup vs baseline: 1.7004x; 1.5371x over previous
import functools

import jax
import jax.numpy as jnp
from jax import lax
from jax.experimental import pallas as pl
from jax.experimental.pallas import tpu as pltpu
from jax.experimental.pallas import tpu_sc as plsc

_N_U = 50000
_N_I = 50000
_D = 128
_NNZ = 600000
_L = 3
_B = 4096

_NPAD = 50688
_NCHUNK = 6
_RCHUNK = 8448
_STRIPE = 528
_TRASH = 8448
_ZROWS = 8456


_EB = 384
_NNZPAD = 602112
_NBLK = 98
_SB = 128
_NSUB = 2
_SCAP = 512


def _spmm_sc(rows, cols, vals, x):
    mesh = plsc.VectorSubcoreMesh(core_axis_name="c", subcore_axis_name="s")

    @functools.partial(
        pl.kernel,
        out_type=jax.ShapeDtypeStruct((_NPAD, _D), jnp.float32),
        mesh=mesh,
        compiler_params=pltpu.CompilerParams(needs_layout_passes=False),
        scratch_types=[
            pltpu.VMEM((_EB,), jnp.int32),
            pltpu.VMEM((_EB,), jnp.int32),
            pltpu.VMEM((_EB,), jnp.float32),
            pltpu.VMEM((_EB,), jnp.int32),
            pltpu.VMEM((_EB,), jnp.int32),
            pltpu.VMEM((_EB,), jnp.float32),
            pltpu.VMEM((_SCAP,), jnp.int32),
            pltpu.VMEM((_SCAP,), jnp.int32),
            pltpu.VMEM((_SCAP,), jnp.float32),
            pltpu.VMEM((_NSUB, _SB), jnp.int32),
            pltpu.VMEM((_NSUB, _SB), jnp.int32),
            pltpu.VMEM((_NSUB, _SB), jnp.float32),
            pltpu.VMEM((_NSUB * _SB, _D), jnp.float32),
            pltpu.VMEM_SHARED((_ZROWS, _D), jnp.float32),
            pltpu.SMEM((1,), jnp.int32),
            pltpu.SMEM((1,), jnp.int32),
            pltpu.SMEM((1,), jnp.int32),
            pltpu.SemaphoreType.DMA,
            pltpu.SemaphoreType.DMA,
            pltpu.SemaphoreType.DMA,
            pltpu.SemaphoreType.DMA,
        ],
    )
    def body(rows_hbm, cols_hbm, vals_hbm, x_hbm, z_hbm,
             rows_a, cols_a, vals_a, rows_b, cols_b, vals_b,
             st_ridx, st_cols, st_vals, ridx2d, cols2d, vals2d, gat_v, zacc,
             off_s, head_s, tail_s, esem_a, esem_b, gsem, ssem):
        c = lax.axis_index("c")
        s = lax.axis_index("s")
        i16 = lax.iota(jnp.int32, 16)
        z16 = jnp.zeros((16,), jnp.float32)
        tile_e0 = s * (_NBLK * _EB)

        def load_blk(bi, rv, cv, vv, sem):
            e0 = tile_e0 + bi * _EB
            pltpu.async_copy(rows_hbm.at[pl.ds(e0, _EB)], rv, sem)
            pltpu.async_copy(cols_hbm.at[pl.ds(e0, _EB)], cv, sem)
            pltpu.async_copy(vals_hbm.at[pl.ds(e0, _EB)], vv, sem)

        def wait_blk(bi, rv, cv, vv, sem):
            e0 = tile_e0 + bi * _EB
            pltpu.make_async_copy(rows_hbm.at[pl.ds(e0, _EB)], rv, sem).wait()
            pltpu.make_async_copy(cols_hbm.at[pl.ds(e0, _EB)], cv, sem).wait()
            pltpu.make_async_copy(vals_hbm.at[pl.ds(e0, _EB)], vv, sem).wait()

        def wait_scatter(slot):
            pltpu.make_async_copy(gat_v.at[pl.ds(slot * _SB, _SB)],
                                  zacc.at[ridx2d.at[slot]], ssem).wait()

        def fin_one():
            head = head_s[0]
            slot = lax.rem(head, _NSUB)
            gb = gat_v.at[pl.ds(slot * _SB, _SB)]
            pltpu.make_async_copy(x_hbm.at[cols2d.at[slot]], gb, gsem).wait()

            @pl.loop(0, _SB // 16)
            def _(g):
                vvec = vals2d.at[slot, pl.ds(g * 16, 16)][...]
                for j2 in range(16):
                    v = vvec[j2]
                    for k in range(_D // 16):
                        sl2 = pl.ds(k * 16, 16)
                        row = gat_v.at[slot * _SB + g * 16 + j2, sl2]
                        row[...] = row[...] * v

            pltpu.async_copy(gat_v.at[pl.ds(slot * _SB, _SB)],
                             zacc.at[ridx2d.at[slot]], ssem, add=True)
            head_s[0] = head + 1

        def issue_one(qoff):
            tail = tail_s[0]

            @pl.when(tail - head_s[0] == _NSUB)
            def _():
                fin_one()

            slot = lax.rem(tail, _NSUB)

            @pl.when(tail >= _NSUB)
            def _():
                wait_scatter(slot)

            @pl.loop(0, _SB // 16)
            def _(g):
                gsl = pl.ds(g * 16, 16)
                ssl = pl.ds(qoff + g * 16, 16)
                cols2d.at[slot, gsl][...] = st_cols.at[ssl][...]
                ridx2d.at[slot, gsl][...] = st_ridx.at[ssl][...]
                vals2d.at[slot, gsl][...] = st_vals.at[ssl][...]
            pltpu.async_copy(x_hbm.at[cols2d.at[slot]],
                             gat_v.at[pl.ds(slot * _SB, _SB)], gsem)
            tail_s[0] = tail + 1

        def drain_all():
            lax.fori_loop(head_s[0], tail_s[0],
                          lambda q, _: (fin_one(), None)[1], None)
            tail = tail_s[0]
            lo = jnp.maximum(tail - _NSUB, 0)
            lax.fori_loop(lo, tail,
                          lambda k, _: (wait_scatter(lax.rem(k, _NSUB)),
                                        None)[1], None)

        def process_blk(bi, rv, cv, vv, esem, base):
            wait_blk(bi, rv, cv, vv, esem)

            def cgroup(g, o):
                sl = pl.ds(g * 16, 16)
                rb = rv.at[sl][...] - base
                inb = lax.bitcast_convert_type(rb, jnp.uint32) < _RCHUNK
                osl = pl.ds(o, 16)
                plsc.store_compressed(st_ridx.at[osl], rb, mask=inb)
                plsc.store_compressed(st_cols.at[osl], cv.at[sl][...],
                                      mask=inb)
                plsc.store_compressed(st_vals.at[osl], vv.at[sl][...],
                                      mask=inb)
                cnt = plsc.all_reduce_population_count(inb)
                return o + cnt[0]

            t = lax.fori_loop(0, _EB // 16, cgroup, off_s[0])
            nbf = lax.div(t, _SB)

            lax.fori_loop(0, nbf,
                          lambda q, _: (issue_one(q * _SB), None)[1], None)

            rem = nbf * _SB
            for g in range(_SB // 16):
                dsl = pl.ds(g * 16, 16)
                ssl = pl.ds(rem + g * 16, 16)
                st_ridx.at[dsl][...] = st_ridx.at[ssl][...]
                st_cols.at[dsl][...] = st_cols.at[ssl][...]
                st_vals.at[dsl][...] = st_vals.at[ssl][...]
            off_s[0] = t - rem

        @pl.loop(0, _NCHUNK // 2)
        def _(p):
            base = (c * (_NCHUNK // 2) + p) * _RCHUNK

            @pl.loop(0, _SB)
            def _(r):
                for k in range(_D // 16):
                    gat_v.at[r, pl.ds(k * 16, 16)][...] = z16
            for i in range(4):
                pltpu.async_copy(gat_v.at[pl.ds(0, _SB)],
                                 zacc.at[pl.ds(s * _STRIPE + i * _SB, _SB)],
                                 esem_a)
            pltpu.sync_copy(gat_v.at[pl.ds(0, 16)],
                            zacc.at[pl.ds(s * _STRIPE + 512, 16)])
            for i in range(4):
                pltpu.make_async_copy(
                    gat_v.at[pl.ds(0, _SB)],
                    zacc.at[pl.ds(s * _STRIPE + i * _SB, _SB)],
                    esem_a).wait()
            plsc.subcore_barrier()
            off_s[0] = 0
            head_s[0] = 0
            tail_s[0] = 0

            load_blk(0, rows_a, cols_a, vals_a, esem_a)

            @pl.loop(0, _NBLK // 2)
            def _(i):
                load_blk(2 * i + 1, rows_b, cols_b, vals_b, esem_b)
                process_blk(2 * i, rows_a, cols_a, vals_a, esem_a, base)
                load_blk(lax.rem(2 * i + 2, _NBLK), rows_a, cols_a, vals_a,
                         esem_a)
                process_blk(2 * i + 1, rows_b, cols_b, vals_b, esem_b, base)

            wait_blk(0, rows_a, cols_a, vals_a, esem_a)

            t = off_s[0]
            for g in range(_SB // 16):
                sl = pl.ds(g * 16, 16)
                keep = (i16 + g * 16) < t
                st_ridx.at[sl][...] = jnp.where(keep, st_ridx.at[sl][...],
                                                _TRASH)
                st_cols.at[sl][...] = jnp.where(keep, st_cols.at[sl][...], 0)
                st_vals.at[sl][...] = jnp.where(keep, st_vals.at[sl][...],
                                                0.0)
            issue_one(0)
            drain_all()

            plsc.subcore_barrier()
            pltpu.sync_copy(zacc.at[pl.ds(s * _STRIPE, _STRIPE)],
                            z_hbm.at[pl.ds(base + s * _STRIPE, _STRIPE)])
            plsc.subcore_barrier()

    return body(rows, cols, vals, x)


def _gather3_sc(e_u, e_i, uids, pos, neg):
    mesh = plsc.VectorSubcoreMesh(core_axis_name="c", subcore_axis_name="s")
    per = _B // 32

    @functools.partial(
        pl.kernel,
        out_type=[jax.ShapeDtypeStruct((_B, _D), jnp.float32)] * 3,
        mesh=mesh,
        scratch_types=[
            pltpu.VMEM((per,), jnp.int32),
            pltpu.VMEM((per, _D), jnp.float32),
        ],
    )
    def body(eu_hbm, ei_hbm, uids_hbm, pos_hbm, neg_hbm, ou, op, on,
             idx_v, buf_v):
        c = lax.axis_index("c")
        s = lax.axis_index("s")
        off = (s * 2 + c) * per
        for ih, tab, oh in ((uids_hbm, eu_hbm, ou),
                            (pos_hbm, ei_hbm, op),
                            (neg_hbm, ei_hbm, on)):
            pltpu.sync_copy(ih.at[pl.ds(off, per)], idx_v)
            pltpu.sync_copy(tab.at[idx_v], buf_v)
            pltpu.sync_copy(buf_v, oh.at[pl.ds(off, per)])

    return body(e_u, e_i, uids, pos, neg)


def _tc_update(zp, zn, e, ws, wp, wn, bs, bp, bn):
    bm = 1584
    dn = (((1,), (1,)), ((), ()))

    def body(zp_r, zn_r, e_r, ws_r, wp_r, wn_r, bs_r, bp_r, bn_r, out_r):
        a = zp_r[...]
        b = zn_r[...]
        ee = e_r[...]
        h = (a - b
             + lax.dot_general(a * ee, wp_r[...], dn,
                               preferred_element_type=jnp.float32)
             - lax.dot_general(b * ee, wn_r[...], dn,
                               preferred_element_type=jnp.float32)
             + lax.dot_general(ee, ws_r[...], dn,
                               preferred_element_type=jnp.float32)
             + (bs_r[...] + bp_r[...] - bn_r[...]))
        out_r[...] = jnp.where(h >= 0, h, 0.2 * h)

    return pl.pallas_call(
        body,
        grid=(_NPAD // bm,),
        in_specs=[pl.BlockSpec((bm, _D), lambda i: (i, 0))] * 3
        + [pl.BlockSpec((_D, _D), lambda i: (0, 0))] * 3
        + [pl.BlockSpec((1, _D), lambda i: (0, 0))] * 3,
        out_specs=pl.BlockSpec((bm, _D), lambda i: (i, 0)),
        out_shape=jax.ShapeDtypeStruct((_NPAD, _D), jnp.float32),
    )(zp, zn, e, ws, wp, wn, bs, bp, bn)


def _tc_loss(u_emb, pos_emb, neg_emb):
    def body(u_r, p_r, n_r, ls_r, lr_r, ps_r, ns_r):
        uu = u_r[...]
        pp = p_r[...]
        nn = n_r[...]
        ps = jnp.sum(uu * pp, axis=1)
        ns = jnp.sum(uu * nn, axis=1)
        ps_r[...] = ps
        ns_r[...] = ns
        d = ps - ns
        sig = 1.0 / (1.0 + jnp.exp(-d))
        lg = jnp.clip(jnp.log(sig), -2000.0, 2000.0)
        ls_r[0, 0] = -jnp.mean(lg)
        lr_r[0, 0] = (jnp.sum(uu * uu) + jnp.sum(pp * pp)
                      + jnp.sum(nn * nn))

    return pl.pallas_call(
        body,
        out_shape=[
            jax.ShapeDtypeStruct((1, 1), jnp.float32),
            jax.ShapeDtypeStruct((1, 1), jnp.float32),
            jax.ShapeDtypeStruct((_B,), jnp.float32),
            jax.ShapeDtypeStruct((_B,), jnp.float32),
        ],
        out_specs=[
            pl.BlockSpec(memory_space=pltpu.SMEM),
            pl.BlockSpec(memory_space=pltpu.SMEM),
            pl.BlockSpec(),
            pl.BlockSpec(),
        ],
    )(u_emb, pos_emb, neg_emb)


def kernel(uids, pos, neg, pos_rows, pos_cols, pos_vals,
           neg_rows, neg_cols, neg_vals, E_u_0, E_i_0,
           Wself_w, Wself_b, Wpos_w, Wpos_b, Wneg_w, Wneg_b):
    e_u = jnp.pad(E_u_0, ((0, _NPAD - _N_U), (0, 0)))
    e_i = jnp.pad(E_i_0, ((0, _NPAD - _N_I), (0, 0)))
    padn = _NNZPAD - _NNZ
    pr = jnp.pad(pos_rows, (0, padn))
    pc = jnp.pad(pos_cols, (0, padn))
    pv = jnp.pad(pos_vals, (0, padn))
    nr = jnp.pad(neg_rows, (0, padn))
    nc = jnp.pad(neg_cols, (0, padn))
    nv = jnp.pad(neg_vals, (0, padn))

    for layer in range(_L):
        ws = Wself_w[layer]
        wp = Wpos_w[layer]
        wn = Wneg_w[layer]
        bs = Wself_b[layer].reshape(1, _D)
        bp = Wpos_b[layer].reshape(1, _D)
        bn = Wneg_b[layer].reshape(1, _D)
        z_u_pos = _spmm_sc(pr, pc, pv, e_i)
        z_u_neg = _spmm_sc(nr, nc, nv, e_i)
        z_i_pos = _spmm_sc(pc, pr, pv, e_u)
        e_u_new = _tc_update(z_u_pos, z_u_neg, e_u, ws, wp, wn, bs, bp, bn)
        z_i_neg = _spmm_sc(nc, nr, nv, e_u)
        e_i = _tc_update(z_i_pos, z_i_neg, e_i, ws, wp, wn, bs, bp, bn)
        e_u = e_u_new

    u_emb, pos_emb, neg_emb = _gather3_sc(e_u, e_i, uids, pos, neg)
    ls, lr, ps, ns = _tc_loss(u_emb, pos_emb, neg_emb)
    return (ls[0, 0], lr[0, 0], ps, ns)


# speedup vs baseline: 2.1913x; 1.2887x over previous
import functools

import jax
import jax.numpy as jnp
from jax import lax
from jax.experimental import pallas as pl
from jax.experimental.pallas import tpu as pltpu
from jax.experimental.pallas import tpu_sc as plsc

_N_U = 50000
_N_I = 50000
_D = 128
_NNZ = 600000
_L = 3
_B = 4096

_NPAD = 50176
_NCHUNK = 4
_RCHUNK = 12544
_STRIPE = 784
_TRASH = 12544
_ZROWS = 12552


_EB = 448
_NNZPAD = 602112
_NBLK = 84
_SB = 96
_NSUB = 2
_SCAP = 544


def _spmm_sc(rows, cols, vals, x):
    mesh = plsc.VectorSubcoreMesh(core_axis_name="c", subcore_axis_name="s")

    @functools.partial(
        pl.kernel,
        out_type=jax.ShapeDtypeStruct((_NPAD, _D), jnp.float32),
        mesh=mesh,
        compiler_params=pltpu.CompilerParams(needs_layout_passes=False),
        scratch_types=[
            pltpu.VMEM((_EB,), jnp.int32),
            pltpu.VMEM((_EB,), jnp.int32),
            pltpu.VMEM((_EB,), jnp.float32),
            pltpu.VMEM((_EB,), jnp.int32),
            pltpu.VMEM((_EB,), jnp.int32),
            pltpu.VMEM((_EB,), jnp.float32),
            pltpu.VMEM((_SCAP,), jnp.int32),
            pltpu.VMEM((_SCAP,), jnp.int32),
            pltpu.VMEM((_SCAP,), jnp.float32),
            pltpu.VMEM((_NSUB, _SB), jnp.int32),
            pltpu.VMEM((_NSUB, _SB), jnp.int32),
            pltpu.VMEM((_NSUB, _SB), jnp.float32),
            pltpu.VMEM((_NSUB * _SB, _D), jnp.float32),
            pltpu.VMEM_SHARED((_ZROWS, _D), jnp.float32),
            pltpu.SMEM((1,), jnp.int32),
            pltpu.SMEM((1,), jnp.int32),
            pltpu.SMEM((1,), jnp.int32),
            pltpu.SemaphoreType.DMA,
            pltpu.SemaphoreType.DMA,
            pltpu.SemaphoreType.DMA,
            pltpu.SemaphoreType.DMA,
        ],
    )
    def body(rows_hbm, cols_hbm, vals_hbm, x_hbm, z_hbm,
             rows_a, cols_a, vals_a, rows_b, cols_b, vals_b,
             st_ridx, st_cols, st_vals, ridx2d, cols2d, vals2d, gat_v, zacc,
             off_s, head_s, tail_s, esem_a, esem_b, gsem, ssem):
        c = lax.axis_index("c")
        s = lax.axis_index("s")
        i16 = lax.iota(jnp.int32, 16)
        z16 = jnp.zeros((16,), jnp.float32)
        tile_e0 = s * (_NBLK * _EB)

        def load_blk(bi, rv, cv, vv, sem):
            e0 = tile_e0 + bi * _EB
            pltpu.async_copy(rows_hbm.at[pl.ds(e0, _EB)], rv, sem)
            pltpu.async_copy(cols_hbm.at[pl.ds(e0, _EB)], cv, sem)
            pltpu.async_copy(vals_hbm.at[pl.ds(e0, _EB)], vv, sem)

        def wait_blk(bi, rv, cv, vv, sem):
            e0 = tile_e0 + bi * _EB
            pltpu.make_async_copy(rows_hbm.at[pl.ds(e0, _EB)], rv, sem).wait()
            pltpu.make_async_copy(cols_hbm.at[pl.ds(e0, _EB)], cv, sem).wait()
            pltpu.make_async_copy(vals_hbm.at[pl.ds(e0, _EB)], vv, sem).wait()

        def wait_scatter(slot):
            pltpu.make_async_copy(gat_v.at[pl.ds(slot * _SB, _SB)],
                                  zacc.at[ridx2d.at[slot]], ssem).wait()

        def fin_one():
            head = head_s[0]
            slot = lax.rem(head, _NSUB)
            gb = gat_v.at[pl.ds(slot * _SB, _SB)]
            pltpu.make_async_copy(x_hbm.at[cols2d.at[slot]], gb, gsem).wait()

            @pl.loop(0, _SB // 16)
            def _(g):
                vvec = vals2d.at[slot, pl.ds(g * 16, 16)][...]
                for j2 in range(16):
                    v = vvec[j2]
                    for k in range(_D // 16):
                        sl2 = pl.ds(k * 16, 16)
                        row = gat_v.at[slot * _SB + g * 16 + j2, sl2]
                        row[...] = row[...] * v

            pltpu.async_copy(gat_v.at[pl.ds(slot * _SB, _SB)],
                             zacc.at[ridx2d.at[slot]], ssem, add=True)
            head_s[0] = head + 1

        def issue_one(qoff):
            tail = tail_s[0]

            @pl.when(tail - head_s[0] == _NSUB)
            def _():
                fin_one()

            slot = lax.rem(tail, _NSUB)

            @pl.when(tail >= _NSUB)
            def _():
                wait_scatter(slot)

            @pl.loop(0, _SB // 16)
            def _(g):
                gsl = pl.ds(g * 16, 16)
                ssl = pl.ds(qoff + g * 16, 16)
                cols2d.at[slot, gsl][...] = st_cols.at[ssl][...]
                ridx2d.at[slot, gsl][...] = st_ridx.at[ssl][...]
                vals2d.at[slot, gsl][...] = st_vals.at[ssl][...]
            pltpu.async_copy(x_hbm.at[cols2d.at[slot]],
                             gat_v.at[pl.ds(slot * _SB, _SB)], gsem)
            tail_s[0] = tail + 1

        def drain_all():
            lax.fori_loop(head_s[0], tail_s[0],
                          lambda q, _: (fin_one(), None)[1], None)
            tail = tail_s[0]
            lo = jnp.maximum(tail - _NSUB, 0)
            lax.fori_loop(lo, tail,
                          lambda k, _: (wait_scatter(lax.rem(k, _NSUB)),
                                        None)[1], None)

        def process_blk(bi, rv, cv, vv, esem, base):
            wait_blk(bi, rv, cv, vv, esem)

            def cgroup(g, o):
                sl = pl.ds(g * 16, 16)
                rb = rv.at[sl][...] - base
                inb = lax.bitcast_convert_type(rb, jnp.uint32) < _RCHUNK
                osl = pl.ds(o, 16)
                plsc.store_compressed(st_ridx.at[osl], rb, mask=inb)
                plsc.store_compressed(st_cols.at[osl], cv.at[sl][...],
                                      mask=inb)
                plsc.store_compressed(st_vals.at[osl], vv.at[sl][...],
                                      mask=inb)
                cnt = plsc.all_reduce_population_count(inb)
                return o + cnt[0]

            t = lax.fori_loop(0, _EB // 16, cgroup, off_s[0])
            nbf = lax.div(t, _SB)

            lax.fori_loop(0, nbf,
                          lambda q, _: (issue_one(q * _SB), None)[1], None)

            rem = nbf * _SB
            for g in range(_SB // 16):
                dsl = pl.ds(g * 16, 16)
                ssl = pl.ds(rem + g * 16, 16)
                st_ridx.at[dsl][...] = st_ridx.at[ssl][...]
                st_cols.at[dsl][...] = st_cols.at[ssl][...]
                st_vals.at[dsl][...] = st_vals.at[ssl][...]
            off_s[0] = t - rem

        @pl.loop(0, _NCHUNK // 2)
        def _(p):
            base = (c * (_NCHUNK // 2) + p) * _RCHUNK

            @pl.loop(0, _SB)
            def _(r):
                for k in range(_D // 16):
                    gat_v.at[r, pl.ds(k * 16, 16)][...] = z16
            for i in range(8):
                pltpu.async_copy(gat_v.at[pl.ds(0, _SB)],
                                 zacc.at[pl.ds(s * _STRIPE + i * _SB, _SB)],
                                 esem_a)
            pltpu.sync_copy(gat_v.at[pl.ds(0, 16)],
                            zacc.at[pl.ds(s * _STRIPE + 768, 16)])
            for i in range(8):
                pltpu.make_async_copy(
                    gat_v.at[pl.ds(0, _SB)],
                    zacc.at[pl.ds(s * _STRIPE + i * _SB, _SB)],
                    esem_a).wait()
            plsc.subcore_barrier()
            off_s[0] = 0
            head_s[0] = 0
            tail_s[0] = 0

            load_blk(0, rows_a, cols_a, vals_a, esem_a)

            @pl.loop(0, _NBLK // 2)
            def _(i):
                load_blk(2 * i + 1, rows_b, cols_b, vals_b, esem_b)
                process_blk(2 * i, rows_a, cols_a, vals_a, esem_a, base)
                load_blk(lax.rem(2 * i + 2, _NBLK), rows_a, cols_a, vals_a,
                         esem_a)
                process_blk(2 * i + 1, rows_b, cols_b, vals_b, esem_b, base)

            wait_blk(0, rows_a, cols_a, vals_a, esem_a)

            t = off_s[0]
            for g in range(_SB // 16):
                sl = pl.ds(g * 16, 16)
                keep = (i16 + g * 16) < t
                st_ridx.at[sl][...] = jnp.where(keep, st_ridx.at[sl][...],
                                                _TRASH)
                st_cols.at[sl][...] = jnp.where(keep, st_cols.at[sl][...], 0)
                st_vals.at[sl][...] = jnp.where(keep, st_vals.at[sl][...],
                                                0.0)
            issue_one(0)
            drain_all()

            plsc.subcore_barrier()
            pltpu.sync_copy(zacc.at[pl.ds(s * _STRIPE, _STRIPE)],
                            z_hbm.at[pl.ds(base + s * _STRIPE, _STRIPE)])
            plsc.subcore_barrier()

    return body(rows, cols, vals, x)


def _gather3_sc(e_u, e_i, uids, pos, neg):
    mesh = plsc.VectorSubcoreMesh(core_axis_name="c", subcore_axis_name="s")
    per = _B // 32

    @functools.partial(
        pl.kernel,
        out_type=[jax.ShapeDtypeStruct((_B, _D), jnp.float32)] * 3,
        mesh=mesh,
        scratch_types=[
            pltpu.VMEM((per,), jnp.int32),
            pltpu.VMEM((per, _D), jnp.float32),
        ],
    )
    def body(eu_hbm, ei_hbm, uids_hbm, pos_hbm, neg_hbm, ou, op, on,
             idx_v, buf_v):
        c = lax.axis_index("c")
        s = lax.axis_index("s")
        off = (s * 2 + c) * per
        for ih, tab, oh in ((uids_hbm, eu_hbm, ou),
                            (pos_hbm, ei_hbm, op),
                            (neg_hbm, ei_hbm, on)):
            pltpu.sync_copy(ih.at[pl.ds(off, per)], idx_v)
            pltpu.sync_copy(tab.at[idx_v], buf_v)
            pltpu.sync_copy(buf_v, oh.at[pl.ds(off, per)])

    return body(e_u, e_i, uids, pos, neg)


def _tc_update(zp, zn, e, ws, wp, wn, bs, bp, bn):
    bm = 1568
    dn = (((1,), (1,)), ((), ()))

    def body(zp_r, zn_r, e_r, ws_r, wp_r, wn_r, bs_r, bp_r, bn_r, out_r):
        a = zp_r[...]
        b = zn_r[...]
        ee = e_r[...]
        h = (a - b
             + lax.dot_general(a * ee, wp_r[...], dn,
                               preferred_element_type=jnp.float32)
             - lax.dot_general(b * ee, wn_r[...], dn,
                               preferred_element_type=jnp.float32)
             + lax.dot_general(ee, ws_r[...], dn,
                               preferred_element_type=jnp.float32)
             + (bs_r[...] + bp_r[...] - bn_r[...]))
        out_r[...] = jnp.where(h >= 0, h, 0.2 * h)

    return pl.pallas_call(
        body,
        grid=(_NPAD // bm,),
        in_specs=[pl.BlockSpec((bm, _D), lambda i: (i, 0))] * 3
        + [pl.BlockSpec((_D, _D), lambda i: (0, 0))] * 3
        + [pl.BlockSpec((1, _D), lambda i: (0, 0))] * 3,
        out_specs=pl.BlockSpec((bm, _D), lambda i: (i, 0)),
        out_shape=jax.ShapeDtypeStruct((_NPAD, _D), jnp.float32),
    )(zp, zn, e, ws, wp, wn, bs, bp, bn)


def _tc_loss(u_emb, pos_emb, neg_emb):
    def body(u_r, p_r, n_r, ls_r, lr_r, ps_r, ns_r):
        uu = u_r[...]
        pp = p_r[...]
        nn = n_r[...]
        ps = jnp.sum(uu * pp, axis=1)
        ns = jnp.sum(uu * nn, axis=1)
        ps_r[...] = ps
        ns_r[...] = ns
        d = ps - ns
        sig = 1.0 / (1.0 + jnp.exp(-d))
        lg = jnp.clip(jnp.log(sig), -2000.0, 2000.0)
        ls_r[0, 0] = -jnp.mean(lg)
        lr_r[0, 0] = (jnp.sum(uu * uu) + jnp.sum(pp * pp)
                      + jnp.sum(nn * nn))

    return pl.pallas_call(
        body,
        out_shape=[
            jax.ShapeDtypeStruct((1, 1), jnp.float32),
            jax.ShapeDtypeStruct((1, 1), jnp.float32),
            jax.ShapeDtypeStruct((_B,), jnp.float32),
            jax.ShapeDtypeStruct((_B,), jnp.float32),
        ],
        out_specs=[
            pl.BlockSpec(memory_space=pltpu.SMEM),
            pl.BlockSpec(memory_space=pltpu.SMEM),
            pl.BlockSpec(),
            pl.BlockSpec(),
        ],
    )(u_emb, pos_emb, neg_emb)


def kernel(uids, pos, neg, pos_rows, pos_cols, pos_vals,
           neg_rows, neg_cols, neg_vals, E_u_0, E_i_0,
           Wself_w, Wself_b, Wpos_w, Wpos_b, Wneg_w, Wneg_b):
    e_u = jnp.pad(E_u_0, ((0, _NPAD - _N_U), (0, 0)))
    e_i = jnp.pad(E_i_0, ((0, _NPAD - _N_I), (0, 0)))
    padn = _NNZPAD - _NNZ
    pr = jnp.pad(pos_rows, (0, padn))
    pc = jnp.pad(pos_cols, (0, padn))
    pv = jnp.pad(pos_vals, (0, padn))
    nr = jnp.pad(neg_rows, (0, padn))
    nc = jnp.pad(neg_cols, (0, padn))
    nv = jnp.pad(neg_vals, (0, padn))

    for layer in range(_L):
        ws = Wself_w[layer]
        wp = Wpos_w[layer]
        wn = Wneg_w[layer]
        bs = Wself_b[layer].reshape(1, _D)
        bp = Wpos_b[layer].reshape(1, _D)
        bn = Wneg_b[layer].reshape(1, _D)
        z_u_pos = _spmm_sc(pr, pc, pv, e_i)
        z_u_neg = _spmm_sc(nr, nc, nv, e_i)
        z_i_pos = _spmm_sc(pc, pr, pv, e_u)
        e_u_new = _tc_update(z_u_pos, z_u_neg, e_u, ws, wp, wn, bs, bp, bn)
        z_i_neg = _spmm_sc(nc, nr, nv, e_u)
        e_i = _tc_update(z_i_pos, z_i_neg, e_i, ws, wp, wn, bs, bp, bn)
        e_u = e_u_new

    u_emb, pos_emb, neg_emb = _gather3_sc(e_u, e_i, uids, pos, neg)
    ls, lr, ps, ns = _tc_loss(u_emb, pos_emb, neg_emb)
    return (ls[0, 0], lr[0, 0], ps, ns)
